# baseline (speedup 1.0000x reference)
import dataclasses
import functools

import jax
import jax.numpy as jnp
from jax import lax
from jax.experimental import pallas as pl
from jax.experimental.pallas import tpu as pltpu
from jax.experimental.pallas import tpu_sc as plsc

N = 10_000_000
NUM_BINS = 64
LANES = 16
NSUB = 16

BLK = 32768
NFULL = N // BLK
TAIL = N - NFULL * BLK
TAIL_TILE = NFULL % NSUB
KMAX = (NFULL + NSUB - 1) // NSUB
VPB = BLK // LANES
TAILV = TAIL // LANES
UNROLL = 4
HUNROLL = 16

_cp = pltpu.CompilerParams()
if "needs_layout_passes" in pltpu.CompilerParams.__dataclass_fields__:
    _cp = dataclasses.replace(_cp, needs_layout_passes=False)

_mesh = plsc.VectorSubcoreMesh(core_axis_name="core", subcore_axis_name="subcore")


def _vregs(buf, i, u):
    return buf[pl.ds((i * UNROLL + u) * LANES, LANES)]


def _minmax_block(buf, nv, mmacc):
    inf = jnp.full((LANES,), jnp.inf, jnp.float32)
    ninf = jnp.full((LANES,), -jnp.inf, jnp.float32)

    init = (tuple(inf for _ in range(UNROLL)), tuple(ninf for _ in range(UNROLL)))

    def body(i, carry):
        mns, mxs = carry
        new_mn, new_mx = [], []
        for u in range(UNROLL):
            v = _vregs(buf, i, u)
            new_mn.append(jnp.minimum(mns[u], v))
            new_mx.append(jnp.maximum(mxs[u], v))
        return tuple(new_mn), tuple(new_mx)

    mns, mxs = plsc.parallel_loop(0, nv // UNROLL, 1, unroll=4,
                                  carry=init)(body)
    mn = jnp.minimum(jnp.minimum(mns[0], mns[1]), jnp.minimum(mns[2], mns[3]))
    mx = jnp.maximum(jnp.maximum(mxs[0], mxs[1]), jnp.maximum(mxs[2], mxs[3]))
    mmacc[pl.ds(0, LANES)] = jnp.minimum(mmacc[pl.ds(0, LANES)], mn)
    mmacc[pl.ds(LANES, LANES)] = jnp.maximum(mmacc[pl.ds(LANES, LANES)], mx)


def _hist_block(buf, nv, hist, gmin_v, scale_v):
    laneoff = lax.iota(jnp.int32, LANES) * NUM_BINS
    ones = jnp.full((LANES,), 1.0, jnp.float32)

    @plsc.parallel_loop(0, nv, 1, unroll=HUNROLL)
    def _(i):
        v = buf[pl.ds(i * LANES, LANES)]
        t = (v - gmin_v) * scale_v
        b = jnp.minimum(t.astype(jnp.int32), NUM_BINS - 1)
        plsc.addupdate_scatter(hist, [b + laneoff], ones)


def _stream(hbm, s, bufA, bufB, semA, semB, block_fn):
    def src(k):
        return hbm.at[pl.ds((k * NSUB + s) * BLK, BLK)]

    def valid(k):
        return (k * NSUB + s) < NFULL

    @pl.when(valid(0))
    def _():
        pltpu.async_copy(src(0), bufA, semA)

    for k in range(KMAX):
        buf, sem = (bufA, semA) if k % 2 == 0 else (bufB, semB)
        nbuf, nsem = (bufB, semB) if k % 2 == 0 else (bufA, semA)
        if k + 1 < KMAX:
            @pl.when(valid(k + 1))
            def _(k=k, nbuf=nbuf, nsem=nsem):
                pltpu.async_copy(src(k + 1), nbuf, nsem)

        @pl.when(valid(k))
        def _(k=k, buf=buf, sem=sem):
            pltpu.make_async_copy(src(k), buf, sem).wait()
            block_fn(buf, VPB)

    @pl.when(s == TAIL_TILE)
    def _():
        tsrc = hbm.at[pl.ds(NFULL * BLK, TAIL)]
        tdst = bufA.at[pl.ds(0, TAIL)]
        pltpu.async_copy(tsrc, tdst, semA)
        pltpu.make_async_copy(tsrc, tdst, semA).wait()
        block_fn(bufA, TAILV)


@functools.partial(
    pl.kernel,
    out_type=jax.ShapeDtypeStruct((2, NSUB, 2 * LANES), jnp.float32),
    mesh=_mesh,
    scratch_types=[
        pltpu.VMEM((BLK,), jnp.float32),
        pltpu.VMEM((BLK,), jnp.float32),
        pltpu.VMEM((2 * LANES,), jnp.float32),
        pltpu.SemaphoreType.DMA,
        pltpu.SemaphoreType.DMA,
    ],
    compiler_params=_cp,
)
def _minmax_kernel(pred_hbm, targ_hbm, mm_hbm, bufA, bufB, mmacc, semA, semB):
    c = lax.axis_index("core")
    s = lax.axis_index("subcore")

    def whole(hbm):
        mmacc[pl.ds(0, LANES)] = jnp.full((LANES,), jnp.inf, jnp.float32)
        mmacc[pl.ds(LANES, LANES)] = jnp.full((LANES,), -jnp.inf, jnp.float32)
        _stream(hbm, s, bufA, bufB, semA, semB,
                lambda buf, nv: _minmax_block(buf, nv, mmacc))
        pltpu.sync_copy(mmacc, mm_hbm.at[c].at[s])

    @pl.when(c == 0)
    def _():
        whole(pred_hbm)

    @pl.when(c == 1)
    def _():
        whole(targ_hbm)


def _reduce_minmax(mml):
    mn = mml[0, pl.ds(0, LANES)]
    mx = mml[0, pl.ds(LANES, LANES)]
    for r in range(1, NSUB):
        mn = jnp.minimum(mn, mml[r, pl.ds(0, LANES)])
        mx = jnp.maximum(mx, mml[r, pl.ds(LANES, LANES)])
    return jnp.min(mn), jnp.max(mx)


@functools.partial(
    pl.kernel,
    out_type=jax.ShapeDtypeStruct((2, NSUB, NUM_BINS), jnp.float32),
    mesh=_mesh,
    scratch_types=[
        pltpu.VMEM((BLK,), jnp.float32),
        pltpu.VMEM((BLK,), jnp.float32),
        pltpu.VMEM((LANES * NUM_BINS,), jnp.float32),
        pltpu.VMEM((NUM_BINS,), jnp.float32),
        pltpu.VMEM((NSUB, 2 * LANES), jnp.float32),
        pltpu.SemaphoreType.DMA,
        pltpu.SemaphoreType.DMA,
    ],
    compiler_params=_cp,
)
def _hist_kernel(pred_hbm, targ_hbm, mm_hbm, cnt_hbm,
                 bufA, bufB, hist, cnt64, mml, semA, semB):
    c = lax.axis_index("core")
    s = lax.axis_index("subcore")

    def whole(hbm):
        pltpu.sync_copy(mm_hbm.at[c], mml)
        gmin_s, gmax_s = _reduce_minmax(mml)
        gmin_v = jnp.full((LANES,), gmin_s)
        range_v = jnp.full((LANES,), gmax_s) - gmin_v
        scale_v = jnp.full((LANES,), float(NUM_BINS), jnp.float32) / range_v

        zeros = jnp.zeros((LANES,), jnp.float32)
        for j in range(LANES * NUM_BINS // LANES):
            hist[pl.ds(j * LANES, LANES)] = zeros
        _stream(hbm, s, bufA, bufB, semA, semB,
                lambda buf, nv: _hist_block(buf, nv, hist, gmin_v, scale_v))
        for j in range(NUM_BINS // LANES):
            acc = hist[pl.ds(j * LANES, LANES)]
            for l in range(1, LANES):
                acc = acc + hist[pl.ds(l * NUM_BINS + j * LANES, LANES)]
            cnt64[pl.ds(j * LANES, LANES)] = acc
        pltpu.sync_copy(cnt64, cnt_hbm.at[c].at[s])

    @pl.when(c == 0)
    def _():
        whole(pred_hbm)

    @pl.when(c == 1)
    def _():
        whole(targ_hbm)


@functools.partial(
    pl.kernel,
    out_type=jax.ShapeDtypeStruct((LANES,), jnp.float32),
    mesh=_mesh,
    scratch_types=[
        pltpu.VMEM((2, NSUB, NUM_BINS), jnp.float32),
        pltpu.VMEM((2, NSUB, 2 * LANES), jnp.float32),
        pltpu.VMEM((LANES,), jnp.float32),
    ],
    compiler_params=_cp,
)
def _finalize(mm_hbm, cnt_hbm, out_hbm, cvm, mvm, ovm):
    c = lax.axis_index("core")
    s = lax.axis_index("subcore")

    @pl.when((c == 0) & (s == 0))
    def _():
        pltpu.sync_copy(cnt_hbm, cvm)
        pltpu.sync_copy(mm_hbm, mvm)

        def cdf(a):
            mn = mvm[a, 0, pl.ds(0, LANES)]
            mx = mvm[a, 0, pl.ds(LANES, LANES)]
            for r in range(1, NSUB):
                mn = jnp.minimum(mn, mvm[a, r, pl.ds(0, LANES)])
                mx = jnp.maximum(mx, mvm[a, r, pl.ds(LANES, LANES)])
            gmin_v = jnp.full((LANES,), jnp.min(mn))
            gmax_v = jnp.full((LANES,), jnp.max(mx))
            width_v = (gmax_v - gmin_v) / jnp.full((LANES,), float(NUM_BINS))
            denom_v = jnp.full((LANES,), float(N)) * width_v
            hs = []
            for j in range(NUM_BINS // LANES):
                tot = cvm[a, 0, pl.ds(j * LANES, LANES)]
                for r in range(1, NSUB):
                    tot = tot + cvm[a, r, pl.ds(j * LANES, LANES)]
                hs.append(tot / denom_v)
            s_sum = hs[0] + hs[1] + hs[2] + hs[3]
            tot_v = jnp.full((LANES,), jnp.sum(s_sum))
            pos = tot_v > jnp.zeros((LANES,), jnp.float32)
            hs = [jnp.where(pos, h / tot_v, h) for h in hs]
            out = []
            carry = jnp.float32(0.0)
            for h in hs:
                out.append(lax.cumsum(h) + jnp.full((LANES,), carry))
                carry = carry + jnp.sum(h)
            return out

        pc = cdf(0)
        tc = cdf(1)
        acc = jnp.abs(pc[0] - tc[0])
        for j in range(1, NUM_BINS // LANES):
            acc = acc + jnp.abs(pc[j] - tc[j])
        d = jnp.sum(acc)
        ovm[...] = jnp.full((LANES,), d) * jnp.full((LANES,), 1.0 / NUM_BINS,
                                                    jnp.float32)
        pltpu.sync_copy(ovm, out_hbm)


def kernel(pred_degrees, target_degrees):
    mm = _minmax_kernel(pred_degrees, target_degrees)
    counts = _hist_kernel(pred_degrees, target_degrees, mm)
    dist = _finalize(mm, counts)
    return dist[0]


# speedup vs baseline: 1.0470x; 1.0470x over previous
import dataclasses
import functools

import jax
import jax.numpy as jnp
from jax import lax
from jax.experimental import pallas as pl
from jax.experimental.pallas import tpu as pltpu
from jax.experimental.pallas import tpu_sc as plsc

N = 10_000_000
NUM_BINS = 64
LANES = 16
NSUB = 16

BLK = 32768
NFULL = N // BLK
TAIL = N - NFULL * BLK
TAIL_TILE = NFULL % NSUB
KMAX = (NFULL + NSUB - 1) // NSUB
VPB = BLK // LANES
TAILV = TAIL // LANES
UNROLL = 4
HUNROLL = 4

_cp = pltpu.CompilerParams()
if "needs_layout_passes" in pltpu.CompilerParams.__dataclass_fields__:
    _cp = dataclasses.replace(_cp, needs_layout_passes=False)

_mesh = plsc.VectorSubcoreMesh(core_axis_name="core", subcore_axis_name="subcore")


def _vregs(buf, i, u):
    return buf[pl.ds((i * UNROLL + u) * LANES, LANES)]


def _minmax_block(buf, nv, mmacc):
    inf = jnp.full((LANES,), jnp.inf, jnp.float32)
    ninf = jnp.full((LANES,), -jnp.inf, jnp.float32)

    init = (tuple(inf for _ in range(UNROLL)), tuple(ninf for _ in range(UNROLL)))

    def body(i, carry):
        mns, mxs = carry
        new_mn, new_mx = [], []
        for u in range(UNROLL):
            v = _vregs(buf, i, u)
            new_mn.append(jnp.minimum(mns[u], v))
            new_mx.append(jnp.maximum(mxs[u], v))
        return tuple(new_mn), tuple(new_mx)

    mns, mxs = plsc.parallel_loop(0, nv // UNROLL, 1, unroll=4,
                                  carry=init)(body)
    mn = jnp.minimum(jnp.minimum(mns[0], mns[1]), jnp.minimum(mns[2], mns[3]))
    mx = jnp.maximum(jnp.maximum(mxs[0], mxs[1]), jnp.maximum(mxs[2], mxs[3]))
    mmacc[pl.ds(0, LANES)] = jnp.minimum(mmacc[pl.ds(0, LANES)], mn)
    mmacc[pl.ds(LANES, LANES)] = jnp.maximum(mmacc[pl.ds(LANES, LANES)], mx)


def _hist_block(buf, nv, hist, gmin_v, scale_v):
    laneoff = lax.iota(jnp.int32, LANES) * NUM_BINS
    ones = jnp.full((LANES,), 1.0, jnp.float32)

    @plsc.parallel_loop(0, nv, 1, unroll=HUNROLL)
    def _(i):
        v = buf[pl.ds(i * LANES, LANES)]
        t = (v - gmin_v) * scale_v
        b = jnp.minimum(t.astype(jnp.int32), NUM_BINS - 1)
        plsc.addupdate_scatter(hist, [b + laneoff], ones)


def _stream(hbm, s, bufA, bufB, semA, semB, block_fn):
    def src(k):
        return hbm.at[pl.ds((k * NSUB + s) * BLK, BLK)]

    def valid(k):
        return (k * NSUB + s) < NFULL

    @pl.when(valid(0))
    def _():
        pltpu.async_copy(src(0), bufA, semA)

    for k in range(KMAX):
        buf, sem = (bufA, semA) if k % 2 == 0 else (bufB, semB)
        nbuf, nsem = (bufB, semB) if k % 2 == 0 else (bufA, semA)
        if k + 1 < KMAX:
            @pl.when(valid(k + 1))
            def _(k=k, nbuf=nbuf, nsem=nsem):
                pltpu.async_copy(src(k + 1), nbuf, nsem)

        @pl.when(valid(k))
        def _(k=k, buf=buf, sem=sem):
            pltpu.make_async_copy(src(k), buf, sem).wait()
            block_fn(buf, VPB)

    @pl.when(s == TAIL_TILE)
    def _():
        tsrc = hbm.at[pl.ds(NFULL * BLK, TAIL)]
        tdst = bufA.at[pl.ds(0, TAIL)]
        pltpu.async_copy(tsrc, tdst, semA)
        pltpu.make_async_copy(tsrc, tdst, semA).wait()
        block_fn(bufA, TAILV)


@functools.partial(
    pl.kernel,
    out_type=jax.ShapeDtypeStruct((2, NSUB, 2 * LANES), jnp.float32),
    mesh=_mesh,
    scratch_types=[
        pltpu.VMEM((BLK,), jnp.float32),
        pltpu.VMEM((BLK,), jnp.float32),
        pltpu.VMEM((2 * LANES,), jnp.float32),
        pltpu.SemaphoreType.DMA,
        pltpu.SemaphoreType.DMA,
    ],
    compiler_params=_cp,
)
def _minmax_kernel(pred_hbm, targ_hbm, mm_hbm, bufA, bufB, mmacc, semA, semB):
    c = lax.axis_index("core")
    s = lax.axis_index("subcore")

    def whole(hbm):
        mmacc[pl.ds(0, LANES)] = jnp.full((LANES,), jnp.inf, jnp.float32)
        mmacc[pl.ds(LANES, LANES)] = jnp.full((LANES,), -jnp.inf, jnp.float32)
        _stream(hbm, s, bufA, bufB, semA, semB,
                lambda buf, nv: _minmax_block(buf, nv, mmacc))
        pltpu.sync_copy(mmacc, mm_hbm.at[c].at[s])

    @pl.when(c == 0)
    def _():
        whole(pred_hbm)

    @pl.when(c == 1)
    def _():
        whole(targ_hbm)


def _reduce_minmax(mml):
    mn = mml[0, pl.ds(0, LANES)]
    mx = mml[0, pl.ds(LANES, LANES)]
    for r in range(1, NSUB):
        mn = jnp.minimum(mn, mml[r, pl.ds(0, LANES)])
        mx = jnp.maximum(mx, mml[r, pl.ds(LANES, LANES)])
    return jnp.min(mn), jnp.max(mx)


@functools.partial(
    pl.kernel,
    out_type=jax.ShapeDtypeStruct((2, NSUB, NUM_BINS), jnp.float32),
    mesh=_mesh,
    scratch_types=[
        pltpu.VMEM((BLK,), jnp.float32),
        pltpu.VMEM((BLK,), jnp.float32),
        pltpu.VMEM((LANES * NUM_BINS,), jnp.float32),
        pltpu.VMEM((NUM_BINS,), jnp.float32),
        pltpu.VMEM((NSUB, 2 * LANES), jnp.float32),
        pltpu.SemaphoreType.DMA,
        pltpu.SemaphoreType.DMA,
    ],
    compiler_params=_cp,
)
def _hist_kernel(pred_hbm, targ_hbm, mm_hbm, cnt_hbm,
                 bufA, bufB, hist, cnt64, mml, semA, semB):
    c = lax.axis_index("core")
    s = lax.axis_index("subcore")

    def whole(hbm):
        pltpu.sync_copy(mm_hbm.at[c], mml)
        gmin_s, gmax_s = _reduce_minmax(mml)
        gmin_v = jnp.full((LANES,), gmin_s)
        range_v = jnp.full((LANES,), gmax_s) - gmin_v
        scale_v = jnp.full((LANES,), float(NUM_BINS), jnp.float32) / range_v

        zeros = jnp.zeros((LANES,), jnp.float32)
        for j in range(LANES * NUM_BINS // LANES):
            hist[pl.ds(j * LANES, LANES)] = zeros
        _stream(hbm, s, bufA, bufB, semA, semB,
                lambda buf, nv: _hist_block(buf, nv, hist, gmin_v, scale_v))
        for j in range(NUM_BINS // LANES):
            acc = hist[pl.ds(j * LANES, LANES)]
            for l in range(1, LANES):
                acc = acc + hist[pl.ds(l * NUM_BINS + j * LANES, LANES)]
            cnt64[pl.ds(j * LANES, LANES)] = acc
        pltpu.sync_copy(cnt64, cnt_hbm.at[c].at[s])

    @pl.when(c == 0)
    def _():
        whole(pred_hbm)

    @pl.when(c == 1)
    def _():
        whole(targ_hbm)


@functools.partial(
    pl.kernel,
    out_type=jax.ShapeDtypeStruct((LANES,), jnp.float32),
    mesh=_mesh,
    scratch_types=[
        pltpu.VMEM((2, NSUB, NUM_BINS), jnp.float32),
        pltpu.VMEM((2, NSUB, 2 * LANES), jnp.float32),
        pltpu.VMEM((LANES,), jnp.float32),
    ],
    compiler_params=_cp,
)
def _finalize(mm_hbm, cnt_hbm, out_hbm, cvm, mvm, ovm):
    c = lax.axis_index("core")
    s = lax.axis_index("subcore")

    @pl.when((c == 0) & (s == 0))
    def _():
        pltpu.sync_copy(cnt_hbm, cvm)
        pltpu.sync_copy(mm_hbm, mvm)

        def cdf(a):
            mn = mvm[a, 0, pl.ds(0, LANES)]
            mx = mvm[a, 0, pl.ds(LANES, LANES)]
            for r in range(1, NSUB):
                mn = jnp.minimum(mn, mvm[a, r, pl.ds(0, LANES)])
                mx = jnp.maximum(mx, mvm[a, r, pl.ds(LANES, LANES)])
            gmin_v = jnp.full((LANES,), jnp.min(mn))
            gmax_v = jnp.full((LANES,), jnp.max(mx))
            width_v = (gmax_v - gmin_v) / jnp.full((LANES,), float(NUM_BINS))
            denom_v = jnp.full((LANES,), float(N)) * width_v
            hs = []
            for j in range(NUM_BINS // LANES):
                tot = cvm[a, 0, pl.ds(j * LANES, LANES)]
                for r in range(1, NSUB):
                    tot = tot + cvm[a, r, pl.ds(j * LANES, LANES)]
                hs.append(tot / denom_v)
            s_sum = hs[0] + hs[1] + hs[2] + hs[3]
            tot_v = jnp.full((LANES,), jnp.sum(s_sum))
            pos = tot_v > jnp.zeros((LANES,), jnp.float32)
            hs = [jnp.where(pos, h / tot_v, h) for h in hs]
            out = []
            carry = jnp.float32(0.0)
            for h in hs:
                out.append(lax.cumsum(h) + jnp.full((LANES,), carry))
                carry = carry + jnp.sum(h)
            return out

        pc = cdf(0)
        tc = cdf(1)
        acc = jnp.abs(pc[0] - tc[0])
        for j in range(1, NUM_BINS // LANES):
            acc = acc + jnp.abs(pc[j] - tc[j])
        d = jnp.sum(acc)
        ovm[...] = jnp.full((LANES,), d) * jnp.full((LANES,), 1.0 / NUM_BINS,
                                                    jnp.float32)
        pltpu.sync_copy(ovm, out_hbm)


def kernel(pred_degrees, target_degrees):
    mm = _minmax_kernel(pred_degrees, target_degrees)
    counts = _hist_kernel(pred_degrees, target_degrees, mm)
    dist = _finalize(mm, counts)
    return dist[0]


# speedup vs baseline: 1.0745x; 1.0262x over previous
import dataclasses
import functools

import jax
import jax.numpy as jnp
from jax import lax
from jax.experimental import pallas as pl
from jax.experimental.pallas import tpu as pltpu
from jax.experimental.pallas import tpu_sc as plsc

N = 10_000_000
NUM_BINS = 64
LANES = 16
NSUB = 16

BLK = 32768
NFULL = N // BLK
TAIL = N - NFULL * BLK
TAIL_TILE = NFULL % NSUB
KMAX = (NFULL + NSUB - 1) // NSUB
VPB = BLK // LANES
TAILV = TAIL // LANES
UNROLL = 4
HUNROLL = 4

_cp = pltpu.CompilerParams()
if "needs_layout_passes" in pltpu.CompilerParams.__dataclass_fields__:
    _cp = dataclasses.replace(_cp, needs_layout_passes=False)

_mesh = plsc.VectorSubcoreMesh(core_axis_name="core", subcore_axis_name="subcore")


def _vregs(buf, i, u):
    return buf[pl.ds((i * UNROLL + u) * LANES, LANES)]


def _minmax_block(buf, nv, mmacc):
    inf = jnp.full((LANES,), jnp.inf, jnp.float32)
    ninf = jnp.full((LANES,), -jnp.inf, jnp.float32)

    init = (tuple(inf for _ in range(UNROLL)), tuple(ninf for _ in range(UNROLL)))

    def body(i, carry):
        mns, mxs = carry
        new_mn, new_mx = [], []
        for u in range(UNROLL):
            v = _vregs(buf, i, u)
            new_mn.append(jnp.minimum(mns[u], v))
            new_mx.append(jnp.maximum(mxs[u], v))
        return tuple(new_mn), tuple(new_mx)

    mns, mxs = plsc.parallel_loop(0, nv // UNROLL, 1, unroll=4,
                                  carry=init)(body)
    mn = jnp.minimum(jnp.minimum(mns[0], mns[1]), jnp.minimum(mns[2], mns[3]))
    mx = jnp.maximum(jnp.maximum(mxs[0], mxs[1]), jnp.maximum(mxs[2], mxs[3]))
    mmacc[pl.ds(0, LANES)] = jnp.minimum(mmacc[pl.ds(0, LANES)], mn)
    mmacc[pl.ds(LANES, LANES)] = jnp.maximum(mmacc[pl.ds(LANES, LANES)], mx)


def _hist_block(buf, nv, hist, gmin_v, scale_v):
    laneoff = lax.iota(jnp.int32, LANES) * NUM_BINS
    ones = jnp.full((LANES,), 1.0, jnp.float32)

    @plsc.parallel_loop(0, nv, 1, unroll=HUNROLL)
    def _(i):
        v = buf[pl.ds(i * LANES, LANES)]
        t = (v - gmin_v) * scale_v
        b = jnp.minimum(t.astype(jnp.int32), NUM_BINS - 1)
        plsc.addupdate_scatter(hist, [b + laneoff], ones)


def _stream(hbm, s, bufA, bufB, semA, semB, block_fn):
    def src(k):
        return hbm.at[pl.ds((k * NSUB + s) * BLK, BLK)]

    def valid(k):
        return (k * NSUB + s) < NFULL

    @pl.when(valid(0))
    def _():
        pltpu.async_copy(src(0), bufA, semA)

    for k in range(KMAX):
        buf, sem = (bufA, semA) if k % 2 == 0 else (bufB, semB)
        nbuf, nsem = (bufB, semB) if k % 2 == 0 else (bufA, semA)
        if k + 1 < KMAX:
            @pl.when(valid(k + 1))
            def _(k=k, nbuf=nbuf, nsem=nsem):
                pltpu.async_copy(src(k + 1), nbuf, nsem)

        @pl.when(valid(k))
        def _(k=k, buf=buf, sem=sem):
            pltpu.make_async_copy(src(k), buf, sem).wait()
            block_fn(buf, VPB)

    @pl.when(s == TAIL_TILE)
    def _():
        tsrc = hbm.at[pl.ds(NFULL * BLK, TAIL)]
        tdst = bufA.at[pl.ds(0, TAIL)]
        pltpu.async_copy(tsrc, tdst, semA)
        pltpu.make_async_copy(tsrc, tdst, semA).wait()
        block_fn(bufA, TAILV)


def _reduce_minmax(mml):
    mn = mml[0, pl.ds(0, LANES)]
    mx = mml[0, pl.ds(LANES, LANES)]
    for r in range(1, NSUB):
        mn = jnp.minimum(mn, mml[r, pl.ds(0, LANES)])
        mx = jnp.maximum(mx, mml[r, pl.ds(LANES, LANES)])
    return jnp.min(mn), jnp.max(mx)


@functools.partial(
    pl.kernel,
    out_type=(
        jax.ShapeDtypeStruct((2, NSUB, NUM_BINS), jnp.float32),
        jax.ShapeDtypeStruct((2, NSUB, 2 * LANES), jnp.float32),
    ),
    mesh=_mesh,
    scratch_types=[
        pltpu.VMEM((BLK,), jnp.float32),
        pltpu.VMEM((BLK,), jnp.float32),
        pltpu.VMEM((LANES * NUM_BINS,), jnp.float32),
        pltpu.VMEM((NUM_BINS,), jnp.float32),
        pltpu.VMEM((NSUB, 2 * LANES), jnp.float32),
        pltpu.VMEM((2 * LANES,), jnp.float32),
        pltpu.SemaphoreType.DMA,
        pltpu.SemaphoreType.DMA,
    ],
    compiler_params=_cp,
)
def _hist_kernel(pred_hbm, targ_hbm, cnt_hbm, mm_hbm,
                 bufA, bufB, hist, cnt64, mml, mmacc, semA, semB):
    c = lax.axis_index("core")
    s = lax.axis_index("subcore")

    def whole(hbm):
        mmacc[pl.ds(0, LANES)] = jnp.full((LANES,), jnp.inf, jnp.float32)
        mmacc[pl.ds(LANES, LANES)] = jnp.full((LANES,), -jnp.inf, jnp.float32)
        _stream(hbm, s, bufA, bufB, semA, semB,
                lambda buf, nv: _minmax_block(buf, nv, mmacc))
        pltpu.sync_copy(mmacc, mm_hbm.at[c].at[s])
        plsc.subcore_barrier()
        pltpu.sync_copy(mm_hbm.at[c], mml)
        gmin_s, gmax_s = _reduce_minmax(mml)
        gmin_v = jnp.full((LANES,), gmin_s)
        range_v = jnp.full((LANES,), gmax_s) - gmin_v
        scale_v = jnp.full((LANES,), float(NUM_BINS), jnp.float32) / range_v

        zeros = jnp.zeros((LANES,), jnp.float32)
        for j in range(LANES * NUM_BINS // LANES):
            hist[pl.ds(j * LANES, LANES)] = zeros
        _stream(hbm, s, bufA, bufB, semA, semB,
                lambda buf, nv: _hist_block(buf, nv, hist, gmin_v, scale_v))
        for j in range(NUM_BINS // LANES):
            acc = hist[pl.ds(j * LANES, LANES)]
            for l in range(1, LANES):
                acc = acc + hist[pl.ds(l * NUM_BINS + j * LANES, LANES)]
            cnt64[pl.ds(j * LANES, LANES)] = acc
        pltpu.sync_copy(cnt64, cnt_hbm.at[c].at[s])

    @pl.when(c == 0)
    def _():
        whole(pred_hbm)

    @pl.when(c == 1)
    def _():
        whole(targ_hbm)


@functools.partial(
    pl.kernel,
    out_type=jax.ShapeDtypeStruct((LANES,), jnp.float32),
    mesh=_mesh,
    scratch_types=[
        pltpu.VMEM((2, NSUB, NUM_BINS), jnp.float32),
        pltpu.VMEM((2, NSUB, 2 * LANES), jnp.float32),
        pltpu.VMEM((LANES,), jnp.float32),
    ],
    compiler_params=_cp,
)
def _finalize(mm_hbm, cnt_hbm, out_hbm, cvm, mvm, ovm):
    c = lax.axis_index("core")
    s = lax.axis_index("subcore")

    @pl.when((c == 0) & (s == 0))
    def _():
        pltpu.sync_copy(cnt_hbm, cvm)
        pltpu.sync_copy(mm_hbm, mvm)

        def cdf(a):
            mn = mvm[a, 0, pl.ds(0, LANES)]
            mx = mvm[a, 0, pl.ds(LANES, LANES)]
            for r in range(1, NSUB):
                mn = jnp.minimum(mn, mvm[a, r, pl.ds(0, LANES)])
                mx = jnp.maximum(mx, mvm[a, r, pl.ds(LANES, LANES)])
            gmin_v = jnp.full((LANES,), jnp.min(mn))
            gmax_v = jnp.full((LANES,), jnp.max(mx))
            width_v = (gmax_v - gmin_v) / jnp.full((LANES,), float(NUM_BINS))
            denom_v = jnp.full((LANES,), float(N)) * width_v
            hs = []
            for j in range(NUM_BINS // LANES):
                tot = cvm[a, 0, pl.ds(j * LANES, LANES)]
                for r in range(1, NSUB):
                    tot = tot + cvm[a, r, pl.ds(j * LANES, LANES)]
                hs.append(tot / denom_v)
            s_sum = hs[0] + hs[1] + hs[2] + hs[3]
            tot_v = jnp.full((LANES,), jnp.sum(s_sum))
            pos = tot_v > jnp.zeros((LANES,), jnp.float32)
            hs = [jnp.where(pos, h / tot_v, h) for h in hs]
            out = []
            carry = jnp.float32(0.0)
            for h in hs:
                out.append(lax.cumsum(h) + jnp.full((LANES,), carry))
                carry = carry + jnp.sum(h)
            return out

        pc = cdf(0)
        tc = cdf(1)
        acc = jnp.abs(pc[0] - tc[0])
        for j in range(1, NUM_BINS // LANES):
            acc = acc + jnp.abs(pc[j] - tc[j])
        d = jnp.sum(acc)
        ovm[...] = jnp.full((LANES,), d) * jnp.full((LANES,), 1.0 / NUM_BINS,
                                                    jnp.float32)
        pltpu.sync_copy(ovm, out_hbm)


def kernel(pred_degrees, target_degrees):
    counts, mm = _hist_kernel(pred_degrees, target_degrees)
    dist = _finalize(mm, counts)
    return dist[0]


# speedup vs baseline: 1.0835x; 1.0084x over previous
import dataclasses
import functools

import jax
import jax.numpy as jnp
from jax import lax
from jax.experimental import pallas as pl
from jax.experimental.pallas import tpu as pltpu
from jax.experimental.pallas import tpu_sc as plsc

N = 10_000_000
NUM_BINS = 64
LANES = 16
NSUB = 16

BLK = 32768
NFULL = N // BLK
TAIL = N - NFULL * BLK
TAIL_TILE = NFULL % NSUB
KMAX = (NFULL + NSUB - 1) // NSUB
VPB = BLK // LANES
TAILV = TAIL // LANES
UNROLL = 4
HUNROLL = 4

_cp = pltpu.CompilerParams()
if "needs_layout_passes" in pltpu.CompilerParams.__dataclass_fields__:
    _cp = dataclasses.replace(_cp, needs_layout_passes=False)

_mesh = plsc.VectorSubcoreMesh(core_axis_name="core", subcore_axis_name="subcore")


def _vregs(buf, i, u):
    return buf[pl.ds((i * UNROLL + u) * LANES, LANES)]


def _minmax_block(buf, nv, mmacc):
    inf = jnp.full((LANES,), jnp.inf, jnp.float32)
    ninf = jnp.full((LANES,), -jnp.inf, jnp.float32)

    init = (tuple(inf for _ in range(UNROLL)), tuple(ninf for _ in range(UNROLL)))

    def body(i, carry):
        mns, mxs = carry
        new_mn, new_mx = [], []
        for u in range(UNROLL):
            v = _vregs(buf, i, u)
            new_mn.append(jnp.minimum(mns[u], v))
            new_mx.append(jnp.maximum(mxs[u], v))
        return tuple(new_mn), tuple(new_mx)

    mns, mxs = plsc.parallel_loop(0, nv // UNROLL, 1, unroll=4,
                                  carry=init)(body)
    mn = jnp.minimum(jnp.minimum(mns[0], mns[1]), jnp.minimum(mns[2], mns[3]))
    mx = jnp.maximum(jnp.maximum(mxs[0], mxs[1]), jnp.maximum(mxs[2], mxs[3]))
    mmacc[pl.ds(0, LANES)] = jnp.minimum(mmacc[pl.ds(0, LANES)], mn)
    mmacc[pl.ds(LANES, LANES)] = jnp.maximum(mmacc[pl.ds(LANES, LANES)], mx)


ROW = NUM_BINS + 1


def _hist_block(buf, nv, hist, off_v, scale_v):
    laneoff = lax.iota(jnp.int32, LANES) * ROW
    ones = jnp.full((LANES,), 1.0, jnp.float32)

    @plsc.parallel_loop(0, nv, 1, unroll=HUNROLL)
    def _(i):
        v = buf[pl.ds(i * LANES, LANES)]
        t = v * scale_v - off_v
        b = t.astype(jnp.int32)
        plsc.addupdate_scatter(hist, [b + laneoff], ones)


def _stream(hbm, s, bufA, bufB, semA, semB, block_fn):
    def src(k):
        return hbm.at[pl.ds((k * NSUB + s) * BLK, BLK)]

    def valid(k):
        return (k * NSUB + s) < NFULL

    @pl.when(valid(0))
    def _():
        pltpu.async_copy(src(0), bufA, semA)

    for k in range(KMAX):
        buf, sem = (bufA, semA) if k % 2 == 0 else (bufB, semB)
        nbuf, nsem = (bufB, semB) if k % 2 == 0 else (bufA, semA)
        if k + 1 < KMAX:
            @pl.when(valid(k + 1))
            def _(k=k, nbuf=nbuf, nsem=nsem):
                pltpu.async_copy(src(k + 1), nbuf, nsem)

        @pl.when(valid(k))
        def _(k=k, buf=buf, sem=sem):
            pltpu.make_async_copy(src(k), buf, sem).wait()
            block_fn(buf, VPB)

    @pl.when(s == TAIL_TILE)
    def _():
        tsrc = hbm.at[pl.ds(NFULL * BLK, TAIL)]
        tdst = bufA.at[pl.ds(0, TAIL)]
        pltpu.async_copy(tsrc, tdst, semA)
        pltpu.make_async_copy(tsrc, tdst, semA).wait()
        block_fn(bufA, TAILV)


def _reduce_minmax(mml):
    mn = mml[0, pl.ds(0, LANES)]
    mx = mml[0, pl.ds(LANES, LANES)]
    for r in range(1, NSUB):
        mn = jnp.minimum(mn, mml[r, pl.ds(0, LANES)])
        mx = jnp.maximum(mx, mml[r, pl.ds(LANES, LANES)])
    return jnp.min(mn), jnp.max(mx)


@functools.partial(
    pl.kernel,
    out_type=(
        jax.ShapeDtypeStruct((2, NSUB, NUM_BINS), jnp.float32),
        jax.ShapeDtypeStruct((2, NSUB, 2 * LANES), jnp.float32),
    ),
    mesh=_mesh,
    scratch_types=[
        pltpu.VMEM((BLK,), jnp.float32),
        pltpu.VMEM((BLK,), jnp.float32),
        pltpu.VMEM((LANES * (NUM_BINS + 1),), jnp.float32),
        pltpu.VMEM((NUM_BINS,), jnp.float32),
        pltpu.VMEM((NSUB, 2 * LANES), jnp.float32),
        pltpu.VMEM((2 * LANES,), jnp.float32),
        pltpu.SemaphoreType.DMA,
        pltpu.SemaphoreType.DMA,
    ],
    compiler_params=_cp,
)
def _hist_kernel(pred_hbm, targ_hbm, cnt_hbm, mm_hbm,
                 bufA, bufB, hist, cnt64, mml, mmacc, semA, semB):
    c = lax.axis_index("core")
    s = lax.axis_index("subcore")

    def whole(hbm):
        mmacc[pl.ds(0, LANES)] = jnp.full((LANES,), jnp.inf, jnp.float32)
        mmacc[pl.ds(LANES, LANES)] = jnp.full((LANES,), -jnp.inf, jnp.float32)
        _stream(hbm, s, bufA, bufB, semA, semB,
                lambda buf, nv: _minmax_block(buf, nv, mmacc))
        pltpu.sync_copy(mmacc, mm_hbm.at[c].at[s])
        plsc.subcore_barrier()
        pltpu.sync_copy(mm_hbm.at[c], mml)
        gmin_s, gmax_s = _reduce_minmax(mml)
        gmin_v = jnp.full((LANES,), gmin_s)
        range_v = jnp.full((LANES,), gmax_s) - gmin_v
        scale_v = jnp.full((LANES,), float(NUM_BINS), jnp.float32) / range_v
        off_v = gmin_v * scale_v

        zeros = jnp.zeros((LANES,), jnp.float32)
        for j in range(LANES * ROW // LANES):
            hist[pl.ds(j * LANES, LANES)] = zeros
        _stream(hbm, s, bufA, bufB, semA, semB,
                lambda buf, nv: _hist_block(buf, nv, hist, off_v, scale_v))
        for j in range(NUM_BINS // LANES):
            acc = hist[pl.ds(j * LANES, LANES)]
            for l in range(1, LANES):
                acc = acc + hist[pl.ds(l * ROW + j * LANES, LANES)]
            cnt64[pl.ds(j * LANES, LANES)] = acc
        over = plsc.load_gather(
            hist, [lax.iota(jnp.int32, LANES) * ROW + NUM_BINS])
        over_t = jnp.full((LANES,), jnp.sum(over))
        is_last = lax.iota(jnp.int32, LANES) == (LANES - 1)
        last = cnt64[pl.ds(3 * LANES, LANES)]
        cnt64[pl.ds(3 * LANES, LANES)] = jnp.where(
            is_last, last + over_t, last)
        pltpu.sync_copy(cnt64, cnt_hbm.at[c].at[s])

    @pl.when(c == 0)
    def _():
        whole(pred_hbm)

    @pl.when(c == 1)
    def _():
        whole(targ_hbm)


@functools.partial(
    pl.kernel,
    out_type=jax.ShapeDtypeStruct((LANES,), jnp.float32),
    mesh=_mesh,
    scratch_types=[
        pltpu.VMEM((2, NSUB, NUM_BINS), jnp.float32),
        pltpu.VMEM((2, NSUB, 2 * LANES), jnp.float32),
        pltpu.VMEM((LANES,), jnp.float32),
    ],
    compiler_params=_cp,
)
def _finalize(mm_hbm, cnt_hbm, out_hbm, cvm, mvm, ovm):
    c = lax.axis_index("core")
    s = lax.axis_index("subcore")

    @pl.when((c == 0) & (s == 0))
    def _():
        pltpu.sync_copy(cnt_hbm, cvm)
        pltpu.sync_copy(mm_hbm, mvm)

        def cdf(a):
            mn = mvm[a, 0, pl.ds(0, LANES)]
            mx = mvm[a, 0, pl.ds(LANES, LANES)]
            for r in range(1, NSUB):
                mn = jnp.minimum(mn, mvm[a, r, pl.ds(0, LANES)])
                mx = jnp.maximum(mx, mvm[a, r, pl.ds(LANES, LANES)])
            gmin_v = jnp.full((LANES,), jnp.min(mn))
            gmax_v = jnp.full((LANES,), jnp.max(mx))
            width_v = (gmax_v - gmin_v) / jnp.full((LANES,), float(NUM_BINS))
            denom_v = jnp.full((LANES,), float(N)) * width_v
            hs = []
            for j in range(NUM_BINS // LANES):
                tot = cvm[a, 0, pl.ds(j * LANES, LANES)]
                for r in range(1, NSUB):
                    tot = tot + cvm[a, r, pl.ds(j * LANES, LANES)]
                hs.append(tot / denom_v)
            s_sum = hs[0] + hs[1] + hs[2] + hs[3]
            tot_v = jnp.full((LANES,), jnp.sum(s_sum))
            pos = tot_v > jnp.zeros((LANES,), jnp.float32)
            hs = [jnp.where(pos, h / tot_v, h) for h in hs]
            out = []
            carry = jnp.float32(0.0)
            for h in hs:
                out.append(lax.cumsum(h) + jnp.full((LANES,), carry))
                carry = carry + jnp.sum(h)
            return out

        pc = cdf(0)
        tc = cdf(1)
        acc = jnp.abs(pc[0] - tc[0])
        for j in range(1, NUM_BINS // LANES):
            acc = acc + jnp.abs(pc[j] - tc[j])
        d = jnp.sum(acc)
        ovm[...] = jnp.full((LANES,), d) * jnp.full((LANES,), 1.0 / NUM_BINS,
                                                    jnp.float32)
        pltpu.sync_copy(ovm, out_hbm)


def kernel(pred_degrees, target_degrees):
    counts, mm = _hist_kernel(pred_degrees, target_degrees)
    dist = _finalize(mm, counts)
    return dist[0]
